# baseline (speedup 1.0000x reference)
import jax
import jax.numpy as jnp
from jax.experimental import pallas as pl
from jax.experimental.pallas import tpu as pltpu
from functools import partial

D_MODEL = 1024
D_STATE = 16
D_CONV = 4
D_INNER = 2048
D_HALF = 1024
DT_RANK = 64
L_SEQ = 2048
CHUNK = 64
N_CHUNKS = L_SEQ // CHUNK


def _silu(x):
    return x * (1.0 / (1.0 + jnp.exp(-x)))


def _dwconv(x, w_ref):
    L = x.shape[0]
    C = x.shape[1]
    z1 = jnp.zeros((1, C), jnp.float32)
    z2 = jnp.zeros((2, C), jnp.float32)
    xm1 = jnp.concatenate([z1, x[:-1]], axis=0)
    xp1 = jnp.concatenate([x[1:], z1], axis=0)
    xp2 = jnp.concatenate([x[2:], z2], axis=0)
    return (w_ref[0:1, :] * xm1 + w_ref[1:2, :] * x
            + w_ref[2:3, :] * xp1 + w_ref[3:4, :] * xp2)


def _inproj_kernel(hs_ref, w_ref, o_ref):
    o_ref[...] = jnp.dot(hs_ref[...], w_ref[...],
                         preferred_element_type=jnp.float32)


def _in_proj(hs2d, w_t):
    M = hs2d.shape[0]
    BM = 512
    return pl.pallas_call(
        _inproj_kernel,
        out_shape=jax.ShapeDtypeStruct((M, D_INNER), jnp.float32),
        grid=(M // BM,),
        in_specs=[
            pl.BlockSpec((BM, D_MODEL), lambda i: (i, 0)),
            pl.BlockSpec((D_MODEL, D_INNER), lambda i: (0, 0)),
        ],
        out_specs=pl.BlockSpec((BM, D_INNER), lambda i: (i, 0)),
        compiler_params=pltpu.CompilerParams(
            dimension_semantics=("parallel",),
        ),
        name="ssmb_in_proj",
    )(hs2d, w_t)


def _scan_kernel(x_ref, cwx_ref, wxp_ref, wdt_ref, bdt2_ref, negAT_ref,
                 dp_ref, y_ref, u_s, xdbl_s, dA_s, dBu_s, C3_s, s_ref):
    x = x_ref[0]
    u = _silu(_dwconv(x, cwx_ref))
    u_s[...] = u
    xdbl_s[...] = jnp.dot(u.astype(jnp.bfloat16), wxp_ref[...],
                          preferred_element_type=jnp.float32)
    negAT = -jnp.exp(negAT_ref[...])

    def chunk_body(c, _):
        base = c * CHUNK
        xd = xdbl_s[pl.ds(base, CHUNK), :]
        dt = jnp.dot(xd[:, :DT_RANK], wdt_ref[...],
                     preferred_element_type=jnp.float32)
        dt = dt + bdt2_ref[...]
        delta = jnp.logaddexp(dt, 0.0)
        u_c = u_s[pl.ds(base, CHUNK), :]
        du = delta * u_c
        dA_s[...] = jnp.exp(delta[:, None, :] * negAT[None, :, :])
        Bm = xd[:, DT_RANK:DT_RANK + D_STATE]
        Cm = xd[:, DT_RANK + D_STATE:]
        dBu_s[...] = du[:, None, :] * Bm[:, :, None]
        C3_s[...] = jnp.broadcast_to(Cm[:, :, None], (CHUNK, D_STATE, D_HALF))

        s = s_ref[...]
        for t in range(CHUNK):
            s = s * dA_s[t] + dBu_s[t]
            y_ref[0, pl.ds(base + t, 1), :] = jnp.sum(
                s * C3_s[t], axis=0, keepdims=True)
        s_ref[...] = s
        return ()

    s_ref[...] = jnp.zeros_like(s_ref)
    jax.lax.fori_loop(0, N_CHUNKS, chunk_body, ())
    y_ref[0] = y_ref[0] + u_s[...] * dp_ref[...]


def _scan(x_raw, cwx, wxp_t, wdt_t, bdt2, A_logT, dp):
    B = x_raw.shape[0]
    return pl.pallas_call(
        _scan_kernel,
        out_shape=jax.ShapeDtypeStruct((B, L_SEQ, D_HALF), jnp.float32),
        grid=(B,),
        in_specs=[
            pl.BlockSpec((1, L_SEQ, D_HALF), lambda b: (b, 0, 0)),
            pl.BlockSpec((D_CONV, D_HALF), lambda b: (0, 0)),
            pl.BlockSpec((D_HALF, DT_RANK + 2 * D_STATE), lambda b: (0, 0)),
            pl.BlockSpec((DT_RANK, D_HALF), lambda b: (0, 0)),
            pl.BlockSpec((1, D_HALF), lambda b: (0, 0)),
            pl.BlockSpec((D_STATE, D_HALF), lambda b: (0, 0)),
            pl.BlockSpec((1, D_HALF), lambda b: (0, 0)),
        ],
        out_specs=pl.BlockSpec((1, L_SEQ, D_HALF), lambda b: (b, 0, 0)),
        scratch_shapes=[
            pltpu.VMEM((L_SEQ, D_HALF), jnp.float32),
            pltpu.VMEM((L_SEQ, DT_RANK + 2 * D_STATE), jnp.float32),
            pltpu.VMEM((CHUNK, D_STATE, D_HALF), jnp.float32),
            pltpu.VMEM((CHUNK, D_STATE, D_HALF), jnp.float32),
            pltpu.VMEM((CHUNK, D_STATE, D_HALF), jnp.float32),
            pltpu.VMEM((D_STATE, D_HALF), jnp.float32),
        ],
        compiler_params=pltpu.CompilerParams(
            dimension_semantics=("arbitrary",),
            vmem_limit_bytes=100 * 1024 * 1024,
        ),
        name="ssmb_scan",
    )(x_raw, cwx, wxp_t, wdt_t, bdt2, A_logT, dp)


def _zconv_kernel(z_ref, cwz_ref, o_ref):
    o_ref[0] = _silu(_dwconv(z_ref[0], cwz_ref)).astype(jnp.bfloat16)


def _zconv(z_raw, cwz):
    B = z_raw.shape[0]
    return pl.pallas_call(
        _zconv_kernel,
        out_shape=jax.ShapeDtypeStruct((B, L_SEQ, D_HALF), jnp.bfloat16),
        grid=(B,),
        in_specs=[
            pl.BlockSpec((1, L_SEQ, D_HALF), lambda b: (b, 0, 0)),
            pl.BlockSpec((D_CONV, D_HALF), lambda b: (0, 0)),
        ],
        out_specs=pl.BlockSpec((1, L_SEQ, D_HALF), lambda b: (b, 0, 0)),
        compiler_params=pltpu.CompilerParams(
            dimension_semantics=("parallel",),
        ),
        name="ssmb_z_conv",
    )(z_raw, cwz)


def _outproj_kernel(y_ref, z_ref, w1_ref, w2_ref, o_ref):
    acc = jnp.dot(y_ref[...].astype(jnp.bfloat16), w1_ref[...],
                  preferred_element_type=jnp.float32)
    acc = acc + jnp.dot(z_ref[...], w2_ref[...],
                        preferred_element_type=jnp.float32)
    o_ref[...] = acc


def _out_proj(y2d, z2d, w1_t, w2_t):
    M = y2d.shape[0]
    BM = 512
    return pl.pallas_call(
        _outproj_kernel,
        out_shape=jax.ShapeDtypeStruct((M, D_MODEL), jnp.float32),
        grid=(M // BM,),
        in_specs=[
            pl.BlockSpec((BM, D_HALF), lambda i: (i, 0)),
            pl.BlockSpec((BM, D_HALF), lambda i: (i, 0)),
            pl.BlockSpec((D_HALF, D_MODEL), lambda i: (0, 0)),
            pl.BlockSpec((D_HALF, D_MODEL), lambda i: (0, 0)),
        ],
        out_specs=pl.BlockSpec((BM, D_MODEL), lambda i: (i, 0)),
        compiler_params=pltpu.CompilerParams(
            dimension_semantics=("parallel",),
        ),
        name="ssmb_out_proj",
    )(y2d, z2d, w1_t, w2_t)


def kernel(hidden_states, W_in, conv_x_w, conv_z_w, W_xp, W_dt, b_dt,
           A_log, Dp, W_out):
    B, L, _ = hidden_states.shape
    hs2d = hidden_states.reshape(B * L, D_MODEL).astype(jnp.bfloat16)
    w_in_t = W_in.T.astype(jnp.bfloat16)
    xz = _in_proj(hs2d, w_in_t).reshape(B, L, D_INNER)
    x_raw = xz[:, :, :D_HALF]
    z_raw = xz[:, :, D_HALF:]

    cwx = conv_x_w[:, 0, :].T
    cwz = conv_z_w[:, 0, :].T
    wxp_t = W_xp.T.astype(jnp.bfloat16)
    wdt_t = W_dt.T
    bdt2 = (2.0 * b_dt).reshape(1, D_HALF)
    A_logT = A_log.T
    dp = Dp.reshape(1, D_HALF)

    y = _scan(x_raw, cwx, wxp_t, wdt_t, bdt2, A_logT, dp)
    zs = _zconv(z_raw, cwz)

    w1_t = W_out[:, :D_HALF].T.astype(jnp.bfloat16)
    w2_t = W_out[:, D_HALF:].T.astype(jnp.bfloat16)
    out = _out_proj(y.reshape(B * L, D_HALF), zs.reshape(B * L, D_HALF),
                    w1_t, w2_t)
    return out.reshape(B, L, D_MODEL)


# speedup vs baseline: 1.1745x; 1.1745x over previous
import jax
import jax.numpy as jnp
from jax.experimental import pallas as pl
from jax.experimental.pallas import tpu as pltpu
from functools import partial

D_MODEL = 1024
D_STATE = 16
D_CONV = 4
D_INNER = 2048
D_HALF = 1024
DT_RANK = 64
L_SEQ = 2048
CHUNK = 64
N_CHUNKS = L_SEQ // CHUNK


def _silu(x):
    return x * (1.0 / (1.0 + jnp.exp(-x)))


def _dwconv(x, w_ref):
    L = x.shape[0]
    C = x.shape[1]
    z1 = jnp.zeros((1, C), jnp.float32)
    z2 = jnp.zeros((2, C), jnp.float32)
    xm1 = jnp.concatenate([z1, x[:-1]], axis=0)
    xp1 = jnp.concatenate([x[1:], z1], axis=0)
    xp2 = jnp.concatenate([x[2:], z2], axis=0)
    return (w_ref[0:1, :] * xm1 + w_ref[1:2, :] * x
            + w_ref[2:3, :] * xp1 + w_ref[3:4, :] * xp2)


def _inproj_kernel(hs_ref, w_ref, o_ref):
    o_ref[...] = jnp.dot(hs_ref[...], w_ref[...],
                         preferred_element_type=jnp.float32)


def _in_proj(hs2d, w_t):
    M = hs2d.shape[0]
    BM = 512
    return pl.pallas_call(
        _inproj_kernel,
        out_shape=jax.ShapeDtypeStruct((M, D_INNER), jnp.float32),
        grid=(M // BM,),
        in_specs=[
            pl.BlockSpec((BM, D_MODEL), lambda i: (i, 0)),
            pl.BlockSpec((D_MODEL, D_INNER), lambda i: (0, 0)),
        ],
        out_specs=pl.BlockSpec((BM, D_INNER), lambda i: (i, 0)),
        compiler_params=pltpu.CompilerParams(
            dimension_semantics=("parallel",),
        ),
        name="ssmb_in_proj",
    )(hs2d, w_t)


def _scan_kernel(x_ref, cwx_ref, wxp_ref, wdt_ref, bdt2_ref, negAT_ref,
                 dp_ref, y_ref, u_s, xdbl_s, dA_s, dBu_s, S3_s, s_ref,
                 pmat_s, mask_s):
    x = x_ref[0]
    u = _silu(_dwconv(x, cwx_ref))
    u_s[...] = u
    xdbl_s[...] = jnp.dot(u.astype(jnp.bfloat16), wxp_ref[...],
                          preferred_element_type=jnp.float32)
    negAT = -jnp.exp(negAT_ref[...])

    lane16 = jax.lax.broadcasted_iota(jnp.int32, (D_STATE, CHUNK * D_STATE), 1)
    sub16 = jax.lax.broadcasted_iota(jnp.int32, (D_STATE, CHUNK * D_STATE), 0)
    pmat_s[...] = jnp.where(lane16 % D_STATE == sub16, 1.0, 0.0).astype(
        jnp.bfloat16)
    laneT = jax.lax.broadcasted_iota(jnp.int32, (CHUNK, CHUNK * D_STATE), 1)
    subT = jax.lax.broadcasted_iota(jnp.int32, (CHUNK, CHUNK * D_STATE), 0)
    mask_s[...] = jnp.where(laneT // D_STATE == subT, 1.0, 0.0)

    def chunk_body(c, _):
        base = c * CHUNK
        xd = xdbl_s[pl.ds(base, CHUNK), :]
        dt = jnp.dot(xd[:, :DT_RANK], wdt_ref[...],
                     preferred_element_type=jnp.float32)
        dt = dt + bdt2_ref[...]
        delta = jnp.logaddexp(dt, 0.0)
        u_c = u_s[pl.ds(base, CHUNK), :]
        du = delta * u_c
        dA_s[...] = jnp.exp(delta[:, None, :] * negAT[None, :, :])
        Bm = xd[:, DT_RANK:DT_RANK + D_STATE]
        Cm = xd[:, DT_RANK + D_STATE:]
        dBu_s[...] = du[:, None, :] * Bm[:, :, None]
        ctile = jnp.dot(Cm.astype(jnp.bfloat16), pmat_s[...],
                        preferred_element_type=jnp.float32)
        wc = (ctile * mask_s[...]).astype(jnp.bfloat16)

        def step(t, _):
            s = s_ref[...] * dA_s[t] + dBu_s[t]
            s_ref[...] = s
            S3_s[t] = s.astype(jnp.bfloat16)
            return ()

        jax.lax.fori_loop(0, CHUNK, step, (), unroll=True)
        y_ref[0, pl.ds(base, CHUNK), :] = jnp.dot(
            wc, S3_s[...].reshape(CHUNK * D_STATE, D_HALF),
            preferred_element_type=jnp.float32)
        return ()

    s_ref[...] = jnp.zeros_like(s_ref)
    jax.lax.fori_loop(0, N_CHUNKS, chunk_body, ())
    y_ref[0] = y_ref[0] + u_s[...] * dp_ref[...]


def _scan(x_raw, cwx, wxp_t, wdt_t, bdt2, A_logT, dp):
    B = x_raw.shape[0]
    return pl.pallas_call(
        _scan_kernel,
        out_shape=jax.ShapeDtypeStruct((B, L_SEQ, D_HALF), jnp.float32),
        grid=(B,),
        in_specs=[
            pl.BlockSpec((1, L_SEQ, D_HALF), lambda b: (b, 0, 0)),
            pl.BlockSpec((D_CONV, D_HALF), lambda b: (0, 0)),
            pl.BlockSpec((D_HALF, DT_RANK + 2 * D_STATE), lambda b: (0, 0)),
            pl.BlockSpec((DT_RANK, D_HALF), lambda b: (0, 0)),
            pl.BlockSpec((1, D_HALF), lambda b: (0, 0)),
            pl.BlockSpec((D_STATE, D_HALF), lambda b: (0, 0)),
            pl.BlockSpec((1, D_HALF), lambda b: (0, 0)),
        ],
        out_specs=pl.BlockSpec((1, L_SEQ, D_HALF), lambda b: (b, 0, 0)),
        scratch_shapes=[
            pltpu.VMEM((L_SEQ, D_HALF), jnp.float32),
            pltpu.VMEM((L_SEQ, DT_RANK + 2 * D_STATE), jnp.float32),
            pltpu.VMEM((CHUNK, D_STATE, D_HALF), jnp.float32),
            pltpu.VMEM((CHUNK, D_STATE, D_HALF), jnp.float32),
            pltpu.VMEM((CHUNK, D_STATE, D_HALF), jnp.bfloat16),
            pltpu.VMEM((D_STATE, D_HALF), jnp.float32),
            pltpu.VMEM((D_STATE, CHUNK * D_STATE), jnp.bfloat16),
            pltpu.VMEM((CHUNK, CHUNK * D_STATE), jnp.float32),
        ],
        compiler_params=pltpu.CompilerParams(
            dimension_semantics=("arbitrary",),
            vmem_limit_bytes=100 * 1024 * 1024,
        ),
        name="ssmb_scan",
    )(x_raw, cwx, wxp_t, wdt_t, bdt2, A_logT, dp)


def _zconv_kernel(z_ref, cwz_ref, o_ref):
    o_ref[0] = _silu(_dwconv(z_ref[0], cwz_ref)).astype(jnp.bfloat16)


def _zconv(z_raw, cwz):
    B = z_raw.shape[0]
    return pl.pallas_call(
        _zconv_kernel,
        out_shape=jax.ShapeDtypeStruct((B, L_SEQ, D_HALF), jnp.bfloat16),
        grid=(B,),
        in_specs=[
            pl.BlockSpec((1, L_SEQ, D_HALF), lambda b: (b, 0, 0)),
            pl.BlockSpec((D_CONV, D_HALF), lambda b: (0, 0)),
        ],
        out_specs=pl.BlockSpec((1, L_SEQ, D_HALF), lambda b: (b, 0, 0)),
        compiler_params=pltpu.CompilerParams(
            dimension_semantics=("parallel",),
        ),
        name="ssmb_z_conv",
    )(z_raw, cwz)


def _outproj_kernel(y_ref, z_ref, w1_ref, w2_ref, o_ref):
    acc = jnp.dot(y_ref[...].astype(jnp.bfloat16), w1_ref[...],
                  preferred_element_type=jnp.float32)
    acc = acc + jnp.dot(z_ref[...], w2_ref[...],
                        preferred_element_type=jnp.float32)
    o_ref[...] = acc


def _out_proj(y2d, z2d, w1_t, w2_t):
    M = y2d.shape[0]
    BM = 512
    return pl.pallas_call(
        _outproj_kernel,
        out_shape=jax.ShapeDtypeStruct((M, D_MODEL), jnp.float32),
        grid=(M // BM,),
        in_specs=[
            pl.BlockSpec((BM, D_HALF), lambda i: (i, 0)),
            pl.BlockSpec((BM, D_HALF), lambda i: (i, 0)),
            pl.BlockSpec((D_HALF, D_MODEL), lambda i: (0, 0)),
            pl.BlockSpec((D_HALF, D_MODEL), lambda i: (0, 0)),
        ],
        out_specs=pl.BlockSpec((BM, D_MODEL), lambda i: (i, 0)),
        compiler_params=pltpu.CompilerParams(
            dimension_semantics=("parallel",),
        ),
        name="ssmb_out_proj",
    )(y2d, z2d, w1_t, w2_t)


def kernel(hidden_states, W_in, conv_x_w, conv_z_w, W_xp, W_dt, b_dt,
           A_log, Dp, W_out):
    B, L, _ = hidden_states.shape
    hs2d = hidden_states.reshape(B * L, D_MODEL).astype(jnp.bfloat16)
    w_in_t = W_in.T.astype(jnp.bfloat16)
    xz = _in_proj(hs2d, w_in_t).reshape(B, L, D_INNER)
    x_raw = xz[:, :, :D_HALF]
    z_raw = xz[:, :, D_HALF:]

    cwx = conv_x_w[:, 0, :].T
    cwz = conv_z_w[:, 0, :].T
    wxp_t = W_xp.T.astype(jnp.bfloat16)
    wdt_t = W_dt.T
    bdt2 = (2.0 * b_dt).reshape(1, D_HALF)
    A_logT = A_log.T
    dp = Dp.reshape(1, D_HALF)

    y = _scan(x_raw, cwx, wxp_t, wdt_t, bdt2, A_logT, dp)
    zs = _zconv(z_raw, cwz)

    w1_t = W_out[:, :D_HALF].T.astype(jnp.bfloat16)
    w2_t = W_out[:, D_HALF:].T.astype(jnp.bfloat16)
    out = _out_proj(y.reshape(B * L, D_HALF), zs.reshape(B * L, D_HALF),
                    w1_t, w2_t)
    return out.reshape(B, L, D_MODEL)


# speedup vs baseline: 1.1809x; 1.0055x over previous
import jax
import jax.numpy as jnp
from jax.experimental import pallas as pl
from jax.experimental.pallas import tpu as pltpu
from functools import partial

D_MODEL = 1024
D_STATE = 16
D_CONV = 4
D_INNER = 2048
D_HALF = 1024
DT_RANK = 64
L_SEQ = 2048
CHUNK = 64
N_CHUNKS = L_SEQ // CHUNK


def _silu(x):
    return x * (1.0 / (1.0 + jnp.exp(-x)))


def _dwconv(x, w_ref):
    L = x.shape[0]
    C = x.shape[1]
    z1 = jnp.zeros((1, C), jnp.float32)
    z2 = jnp.zeros((2, C), jnp.float32)
    xm1 = jnp.concatenate([z1, x[:-1]], axis=0)
    xp1 = jnp.concatenate([x[1:], z1], axis=0)
    xp2 = jnp.concatenate([x[2:], z2], axis=0)
    return (w_ref[0:1, :] * xm1 + w_ref[1:2, :] * x
            + w_ref[2:3, :] * xp1 + w_ref[3:4, :] * xp2)


def _inproj_kernel(hs_ref, w_ref, o_ref):
    o_ref[...] = jnp.dot(hs_ref[...], w_ref[...],
                         preferred_element_type=jnp.float32)


def _in_proj(hs2d, w_t):
    M = hs2d.shape[0]
    BM = 1024
    return pl.pallas_call(
        _inproj_kernel,
        out_shape=jax.ShapeDtypeStruct((M, D_INNER), jnp.float32),
        grid=(M // BM,),
        in_specs=[
            pl.BlockSpec((BM, D_MODEL), lambda i: (i, 0)),
            pl.BlockSpec((D_MODEL, D_INNER), lambda i: (0, 0)),
        ],
        out_specs=pl.BlockSpec((BM, D_INNER), lambda i: (i, 0)),
        compiler_params=pltpu.CompilerParams(
            dimension_semantics=("parallel",),
        ),
        name="ssmb_in_proj",
    )(hs2d, w_t)


def _scan_kernel(x_ref, cwx_ref, wxp_ref, wdt_ref, bdt2_ref, negAT_ref,
                 dp_ref, y_ref, u_s, xdbl_s, dA_s, dBu_s, S3_s, s_ref,
                 pmat_s, mask_s):
    x = x_ref[0]
    u = _silu(_dwconv(x, cwx_ref))
    u_s[...] = u
    xdbl_s[...] = jnp.dot(u.astype(jnp.bfloat16), wxp_ref[...],
                          preferred_element_type=jnp.float32)
    negAT = -jnp.exp(negAT_ref[...])

    lane16 = jax.lax.broadcasted_iota(jnp.int32, (D_STATE, CHUNK * D_STATE), 1)
    sub16 = jax.lax.broadcasted_iota(jnp.int32, (D_STATE, CHUNK * D_STATE), 0)
    pmat_s[...] = jnp.where(lane16 % D_STATE == sub16, 1.0, 0.0).astype(
        jnp.bfloat16)
    laneT = jax.lax.broadcasted_iota(jnp.int32, (CHUNK, CHUNK * D_STATE), 1)
    subT = jax.lax.broadcasted_iota(jnp.int32, (CHUNK, CHUNK * D_STATE), 0)
    mask_s[...] = jnp.where(laneT // D_STATE == subT, 1.0, 0.0)

    def chunk_body(c, _):
        base = c * CHUNK
        xd = xdbl_s[pl.ds(base, CHUNK), :]
        dt = jnp.dot(xd[:, :DT_RANK], wdt_ref[...],
                     preferred_element_type=jnp.float32)
        dt = dt + bdt2_ref[...]
        delta = jnp.logaddexp(dt, 0.0)
        u_c = u_s[pl.ds(base, CHUNK), :]
        du = delta * u_c
        dA_s[...] = jnp.exp(delta[:, None, :] * negAT[None, :, :])
        Bm = xd[:, DT_RANK:DT_RANK + D_STATE]
        Cm = xd[:, DT_RANK + D_STATE:]
        dBu_s[...] = du[:, None, :] * Bm[:, :, None]
        ctile = jnp.dot(Cm.astype(jnp.bfloat16), pmat_s[...],
                        preferred_element_type=jnp.float32)
        wc = (ctile * mask_s[...]).astype(jnp.bfloat16)

        def step(t, _):
            s = s_ref[...] * dA_s[t] + dBu_s[t]
            s_ref[...] = s
            S3_s[t] = s.astype(jnp.bfloat16)
            return ()

        jax.lax.fori_loop(0, CHUNK, step, (), unroll=True)
        y_ref[0, pl.ds(base, CHUNK), :] = jnp.dot(
            wc, S3_s[...].reshape(CHUNK * D_STATE, D_HALF),
            preferred_element_type=jnp.float32) + u_c * dp_ref[...]
        return ()

    s_ref[...] = jnp.zeros_like(s_ref)
    jax.lax.fori_loop(0, N_CHUNKS, chunk_body, ())


def _scan(x_raw, cwx, wxp_t, wdt_t, bdt2, A_logT, dp):
    B = x_raw.shape[0]
    return pl.pallas_call(
        _scan_kernel,
        out_shape=jax.ShapeDtypeStruct((B, L_SEQ, D_HALF), jnp.float32),
        grid=(B,),
        in_specs=[
            pl.BlockSpec((1, L_SEQ, D_HALF), lambda b: (b, 0, 0)),
            pl.BlockSpec((D_CONV, D_HALF), lambda b: (0, 0)),
            pl.BlockSpec((D_HALF, DT_RANK + 2 * D_STATE), lambda b: (0, 0)),
            pl.BlockSpec((DT_RANK, D_HALF), lambda b: (0, 0)),
            pl.BlockSpec((1, D_HALF), lambda b: (0, 0)),
            pl.BlockSpec((D_STATE, D_HALF), lambda b: (0, 0)),
            pl.BlockSpec((1, D_HALF), lambda b: (0, 0)),
        ],
        out_specs=pl.BlockSpec((1, L_SEQ, D_HALF), lambda b: (b, 0, 0)),
        scratch_shapes=[
            pltpu.VMEM((L_SEQ, D_HALF), jnp.float32),
            pltpu.VMEM((L_SEQ, DT_RANK + 2 * D_STATE), jnp.float32),
            pltpu.VMEM((CHUNK, D_STATE, D_HALF), jnp.float32),
            pltpu.VMEM((CHUNK, D_STATE, D_HALF), jnp.float32),
            pltpu.VMEM((CHUNK, D_STATE, D_HALF), jnp.bfloat16),
            pltpu.VMEM((D_STATE, D_HALF), jnp.float32),
            pltpu.VMEM((D_STATE, CHUNK * D_STATE), jnp.bfloat16),
            pltpu.VMEM((CHUNK, CHUNK * D_STATE), jnp.float32),
        ],
        compiler_params=pltpu.CompilerParams(
            dimension_semantics=("arbitrary",),
            vmem_limit_bytes=100 * 1024 * 1024,
        ),
        name="ssmb_scan",
    )(x_raw, cwx, wxp_t, wdt_t, bdt2, A_logT, dp)


def _zconv_kernel(z_ref, cwz_ref, o_ref):
    o_ref[0] = _silu(_dwconv(z_ref[0], cwz_ref)).astype(jnp.bfloat16)


def _zconv(z_raw, cwz):
    B = z_raw.shape[0]
    return pl.pallas_call(
        _zconv_kernel,
        out_shape=jax.ShapeDtypeStruct((B, L_SEQ, D_HALF), jnp.bfloat16),
        grid=(B,),
        in_specs=[
            pl.BlockSpec((1, L_SEQ, D_HALF), lambda b: (b, 0, 0)),
            pl.BlockSpec((D_CONV, D_HALF), lambda b: (0, 0)),
        ],
        out_specs=pl.BlockSpec((1, L_SEQ, D_HALF), lambda b: (b, 0, 0)),
        compiler_params=pltpu.CompilerParams(
            dimension_semantics=("parallel",),
        ),
        name="ssmb_z_conv",
    )(z_raw, cwz)


def _outproj_kernel(y_ref, z_ref, w1_ref, w2_ref, o_ref):
    acc = jnp.dot(y_ref[...].astype(jnp.bfloat16), w1_ref[...],
                  preferred_element_type=jnp.float32)
    acc = acc + jnp.dot(z_ref[...], w2_ref[...],
                        preferred_element_type=jnp.float32)
    o_ref[...] = acc


def _out_proj(y2d, z2d, w1_t, w2_t):
    M = y2d.shape[0]
    BM = 1024
    return pl.pallas_call(
        _outproj_kernel,
        out_shape=jax.ShapeDtypeStruct((M, D_MODEL), jnp.float32),
        grid=(M // BM,),
        in_specs=[
            pl.BlockSpec((BM, D_HALF), lambda i: (i, 0)),
            pl.BlockSpec((BM, D_HALF), lambda i: (i, 0)),
            pl.BlockSpec((D_HALF, D_MODEL), lambda i: (0, 0)),
            pl.BlockSpec((D_HALF, D_MODEL), lambda i: (0, 0)),
        ],
        out_specs=pl.BlockSpec((BM, D_MODEL), lambda i: (i, 0)),
        compiler_params=pltpu.CompilerParams(
            dimension_semantics=("parallel",),
        ),
        name="ssmb_out_proj",
    )(y2d, z2d, w1_t, w2_t)


def kernel(hidden_states, W_in, conv_x_w, conv_z_w, W_xp, W_dt, b_dt,
           A_log, Dp, W_out):
    B, L, _ = hidden_states.shape
    hs2d = hidden_states.reshape(B * L, D_MODEL).astype(jnp.bfloat16)
    w_in_t = W_in.T.astype(jnp.bfloat16)
    xz = _in_proj(hs2d, w_in_t).reshape(B, L, D_INNER)
    x_raw = xz[:, :, :D_HALF]
    z_raw = xz[:, :, D_HALF:]

    cwx = conv_x_w[:, 0, :].T
    cwz = conv_z_w[:, 0, :].T
    wxp_t = W_xp.T.astype(jnp.bfloat16)
    wdt_t = W_dt.T
    bdt2 = (2.0 * b_dt).reshape(1, D_HALF)
    A_logT = A_log.T
    dp = Dp.reshape(1, D_HALF)

    y = _scan(x_raw, cwx, wxp_t, wdt_t, bdt2, A_logT, dp)
    zs = _zconv(z_raw, cwz)

    w1_t = W_out[:, :D_HALF].T.astype(jnp.bfloat16)
    w2_t = W_out[:, D_HALF:].T.astype(jnp.bfloat16)
    out = _out_proj(y.reshape(B * L, D_HALF), zs.reshape(B * L, D_HALF),
                    w1_t, w2_t)
    return out.reshape(B, L, D_MODEL)


# speedup vs baseline: 1.3154x; 1.1139x over previous
import jax
import jax.numpy as jnp
from jax.experimental import pallas as pl
from jax.experimental.pallas import tpu as pltpu
from functools import partial

D_MODEL = 1024
D_STATE = 16
D_CONV = 4
D_INNER = 2048
D_HALF = 1024
DT_RANK = 64
L_SEQ = 2048
CHUNK = 64
N_CHUNKS = L_SEQ // CHUNK


def _silu(x):
    return x * (1.0 / (1.0 + jnp.exp(-x)))


def _dwconv(x, w_ref):
    L = x.shape[0]
    C = x.shape[1]
    z1 = jnp.zeros((1, C), jnp.float32)
    z2 = jnp.zeros((2, C), jnp.float32)
    xm1 = jnp.concatenate([z1, x[:-1]], axis=0)
    xp1 = jnp.concatenate([x[1:], z1], axis=0)
    xp2 = jnp.concatenate([x[2:], z2], axis=0)
    return (w_ref[0:1, :] * xm1 + w_ref[1:2, :] * x
            + w_ref[2:3, :] * xp1 + w_ref[3:4, :] * xp2)


def _inproj_kernel(hs_ref, w_ref, x_ref, z_ref):
    xz = jnp.dot(hs_ref[...], w_ref[...],
                 preferred_element_type=jnp.float32)
    x_ref[...] = xz[:, :D_HALF]
    z_ref[...] = xz[:, D_HALF:].astype(jnp.bfloat16)


def _in_proj(hs2d, w_t):
    M = hs2d.shape[0]
    BM = 1024
    return pl.pallas_call(
        _inproj_kernel,
        out_shape=[jax.ShapeDtypeStruct((M, D_HALF), jnp.float32),
                   jax.ShapeDtypeStruct((M, D_HALF), jnp.bfloat16)],
        grid=(M // BM,),
        in_specs=[
            pl.BlockSpec((BM, D_MODEL), lambda i: (i, 0)),
            pl.BlockSpec((D_MODEL, D_INNER), lambda i: (0, 0)),
        ],
        out_specs=[pl.BlockSpec((BM, D_HALF), lambda i: (i, 0)),
                   pl.BlockSpec((BM, D_HALF), lambda i: (i, 0))],
        compiler_params=pltpu.CompilerParams(
            dimension_semantics=("parallel",),
        ),
        name="ssmb_in_proj",
    )(hs2d, w_t)


def _scan_kernel(x_ref, cwx_ref, wxp_ref, wdt_ref, bdt2_ref, negAT_ref,
                 dp_ref, y_ref, u_s, xdbl_s, dA_s, dBu_s, S3_s, s_ref,
                 pmat_s, mask_s):
    x = x_ref[0]
    u = _silu(_dwconv(x, cwx_ref))
    u_s[...] = u
    xdbl_s[...] = jnp.dot(u.astype(jnp.bfloat16), wxp_ref[...],
                          preferred_element_type=jnp.float32)
    negAT = -jnp.exp(negAT_ref[...])

    lane16 = jax.lax.broadcasted_iota(jnp.int32, (D_STATE, CHUNK * D_STATE), 1)
    sub16 = jax.lax.broadcasted_iota(jnp.int32, (D_STATE, CHUNK * D_STATE), 0)
    pmat_s[...] = jnp.where(lane16 % D_STATE == sub16, 1.0, 0.0).astype(
        jnp.bfloat16)
    laneT = jax.lax.broadcasted_iota(jnp.int32, (CHUNK, CHUNK * D_STATE), 1)
    subT = jax.lax.broadcasted_iota(jnp.int32, (CHUNK, CHUNK * D_STATE), 0)
    mask_s[...] = jnp.where(laneT // D_STATE == subT, 1.0, 0.0)

    def chunk_body(c, _):
        base = c * CHUNK
        xd = xdbl_s[pl.ds(base, CHUNK), :]
        dt = jnp.dot(xd[:, :DT_RANK], wdt_ref[...],
                     preferred_element_type=jnp.float32)
        dt = dt + bdt2_ref[...]
        delta = jnp.logaddexp(dt, 0.0)
        u_c = u_s[pl.ds(base, CHUNK), :]
        du = delta * u_c
        dA_s[...] = jnp.exp(delta[:, None, :] * negAT[None, :, :])
        Bm = xd[:, DT_RANK:DT_RANK + D_STATE]
        Cm = xd[:, DT_RANK + D_STATE:]
        dBu_s[...] = du[:, None, :] * Bm[:, :, None]
        ctile = jnp.dot(Cm.astype(jnp.bfloat16), pmat_s[...],
                        preferred_element_type=jnp.float32)
        wc = (ctile * mask_s[...]).astype(jnp.bfloat16)

        def step(t, _):
            s = s_ref[...] * dA_s[t] + dBu_s[t]
            s_ref[...] = s
            S3_s[t] = s.astype(jnp.bfloat16)
            return ()

        jax.lax.fori_loop(0, CHUNK, step, (), unroll=True)
        y_ref[0, pl.ds(base, CHUNK), :] = jnp.dot(
            wc, S3_s[...].reshape(CHUNK * D_STATE, D_HALF),
            preferred_element_type=jnp.float32) + u_c * dp_ref[...]
        return ()

    s_ref[...] = jnp.zeros_like(s_ref)
    jax.lax.fori_loop(0, N_CHUNKS, chunk_body, ())


def _scan(x_raw, cwx, wxp_t, wdt_t, bdt2, A_logT, dp):
    B = x_raw.shape[0]
    return pl.pallas_call(
        _scan_kernel,
        out_shape=jax.ShapeDtypeStruct((B, L_SEQ, D_HALF), jnp.float32),
        grid=(B,),
        in_specs=[
            pl.BlockSpec((1, L_SEQ, D_HALF), lambda b: (b, 0, 0)),
            pl.BlockSpec((D_CONV, D_HALF), lambda b: (0, 0)),
            pl.BlockSpec((D_HALF, DT_RANK + 2 * D_STATE), lambda b: (0, 0)),
            pl.BlockSpec((DT_RANK, D_HALF), lambda b: (0, 0)),
            pl.BlockSpec((1, D_HALF), lambda b: (0, 0)),
            pl.BlockSpec((D_STATE, D_HALF), lambda b: (0, 0)),
            pl.BlockSpec((1, D_HALF), lambda b: (0, 0)),
        ],
        out_specs=pl.BlockSpec((1, L_SEQ, D_HALF), lambda b: (b, 0, 0)),
        scratch_shapes=[
            pltpu.VMEM((L_SEQ, D_HALF), jnp.float32),
            pltpu.VMEM((L_SEQ, DT_RANK + 2 * D_STATE), jnp.float32),
            pltpu.VMEM((CHUNK, D_STATE, D_HALF), jnp.float32),
            pltpu.VMEM((CHUNK, D_STATE, D_HALF), jnp.float32),
            pltpu.VMEM((CHUNK, D_STATE, D_HALF), jnp.bfloat16),
            pltpu.VMEM((D_STATE, D_HALF), jnp.float32),
            pltpu.VMEM((D_STATE, CHUNK * D_STATE), jnp.bfloat16),
            pltpu.VMEM((CHUNK, CHUNK * D_STATE), jnp.float32),
        ],
        compiler_params=pltpu.CompilerParams(
            dimension_semantics=("arbitrary",),
            vmem_limit_bytes=100 * 1024 * 1024,
        ),
        name="ssmb_scan",
    )(x_raw, cwx, wxp_t, wdt_t, bdt2, A_logT, dp)


def _zconv_kernel(z_ref, cwz_ref, o_ref):
    z = z_ref[0].astype(jnp.float32)
    o_ref[0] = _silu(_dwconv(z, cwz_ref)).astype(jnp.bfloat16)


def _zconv(z_raw, cwz):
    B = z_raw.shape[0]
    return pl.pallas_call(
        _zconv_kernel,
        out_shape=jax.ShapeDtypeStruct((B, L_SEQ, D_HALF), jnp.bfloat16),
        grid=(B,),
        in_specs=[
            pl.BlockSpec((1, L_SEQ, D_HALF), lambda b: (b, 0, 0)),
            pl.BlockSpec((D_CONV, D_HALF), lambda b: (0, 0)),
        ],
        out_specs=pl.BlockSpec((1, L_SEQ, D_HALF), lambda b: (b, 0, 0)),
        compiler_params=pltpu.CompilerParams(
            dimension_semantics=("parallel",),
        ),
        name="ssmb_z_conv",
    )(z_raw, cwz)


def _outproj_kernel(y_ref, z_ref, w1_ref, w2_ref, o_ref):
    acc = jnp.dot(y_ref[...].astype(jnp.bfloat16), w1_ref[...],
                  preferred_element_type=jnp.float32)
    acc = acc + jnp.dot(z_ref[...], w2_ref[...],
                        preferred_element_type=jnp.float32)
    o_ref[...] = acc


def _out_proj(y2d, z2d, w1_t, w2_t):
    M = y2d.shape[0]
    BM = 1024
    return pl.pallas_call(
        _outproj_kernel,
        out_shape=jax.ShapeDtypeStruct((M, D_MODEL), jnp.float32),
        grid=(M // BM,),
        in_specs=[
            pl.BlockSpec((BM, D_HALF), lambda i: (i, 0)),
            pl.BlockSpec((BM, D_HALF), lambda i: (i, 0)),
            pl.BlockSpec((D_HALF, D_MODEL), lambda i: (0, 0)),
            pl.BlockSpec((D_HALF, D_MODEL), lambda i: (0, 0)),
        ],
        out_specs=pl.BlockSpec((BM, D_MODEL), lambda i: (i, 0)),
        compiler_params=pltpu.CompilerParams(
            dimension_semantics=("parallel",),
        ),
        name="ssmb_out_proj",
    )(y2d, z2d, w1_t, w2_t)


def kernel(hidden_states, W_in, conv_x_w, conv_z_w, W_xp, W_dt, b_dt,
           A_log, Dp, W_out):
    B, L, _ = hidden_states.shape
    hs2d = hidden_states.reshape(B * L, D_MODEL).astype(jnp.bfloat16)
    w_in_t = W_in.T.astype(jnp.bfloat16)
    x2d, z2d = _in_proj(hs2d, w_in_t)
    x_raw = x2d.reshape(B, L, D_HALF)
    z_raw = z2d.reshape(B, L, D_HALF)

    cwx = conv_x_w[:, 0, :].T
    cwz = conv_z_w[:, 0, :].T
    wxp_t = W_xp.T.astype(jnp.bfloat16)
    wdt_t = W_dt.T
    bdt2 = (2.0 * b_dt).reshape(1, D_HALF)
    A_logT = A_log.T
    dp = Dp.reshape(1, D_HALF)

    y = _scan(x_raw, cwx, wxp_t, wdt_t, bdt2, A_logT, dp)
    zs = _zconv(z_raw, cwz)

    w1_t = W_out[:, :D_HALF].T.astype(jnp.bfloat16)
    w2_t = W_out[:, D_HALF:].T.astype(jnp.bfloat16)
    out = _out_proj(y.reshape(B * L, D_HALF), zs.reshape(B * L, D_HALF),
                    w1_t, w2_t)
    return out.reshape(B, L, D_MODEL)
